# Initial kernel scaffold; baseline (speedup 1.0000x reference)
import jax
import jax.numpy as jnp
from jax.experimental import pallas as pl


def kernel(names):
    raise NotImplementedError("write your pallas kernel here")


# speedup vs baseline: 21.1200x; 21.1200x over previous
import jax
import jax.numpy as jnp
from jax import lax
from jax.experimental import pallas as pl
from jax.experimental.pallas import tpu as pltpu
from jax.experimental.pallas import tpu_sc as plsc
import functools

B, L, C = 16384, 50, 1000
NC, NS = 2, 16
NW = NC * NS
RPW = B // NW
R = 32
NCHUNK = RPW // R
EPC = R * L
VECS = EPC // 16
CB = R * C


def _body(names_hbm, out_hbm, names_v, rowmul_v, buf0, buf1, semn, sem0, sem1):
    wid = lax.axis_index("s") * NC + lax.axis_index("c")
    ebase = wid * (RPW * L)
    obase = wid * (RPW * C)

    names_cp = pltpu.async_copy(
        names_hbm.at[pl.ds(ebase, RPW * L)], names_v, semn)

    zeros = jnp.zeros((16,), jnp.float32)
    ones = jnp.full((16,), 1.0, jnp.float32)
    iota = lax.iota(jnp.int32, 16)

    def zero_body(i, _):
        buf0[pl.ds(i * 16, 16)] = zeros
        buf1[pl.ds(i * 16, 16)] = zeros
        return 0
    lax.fori_loop(0, CB // 16, zero_body, 0, unroll=4)

    def rm_body(j, _):
        e = j * 16 + iota
        rowmul_v[pl.ds(j * 16, 16)] = (e // L) * C
        return 0
    lax.fori_loop(0, VECS, rm_body, 0, unroll=4)

    names_cp.wait()

    def scatter_chunk(buf, c, val_vec):
        coff = c * EPC

        def s_body(j, _):
            nv = names_v[pl.ds(coff + j * 16, 16)]
            rm = rowmul_v[pl.ds(j * 16, 16)]
            plsc.store_scatter(buf, [rm + nv], val_vec)
            return 0
        lax.fori_loop(0, VECS, s_body, 0, unroll=4)

    def fire(buf, c, sem):
        pltpu.async_copy(buf, out_hbm.at[pl.ds(obase + c * CB, CB)], sem)

    def wait_out(buf, c, sem):
        pltpu.make_async_copy(
            buf, out_hbm.at[pl.ds(obase + c * CB, CB)], sem).wait()

    scatter_chunk(buf0, 0, ones)
    fire(buf0, 0, sem0)
    scatter_chunk(buf1, 1, ones)
    fire(buf1, 1, sem1)

    def pair_body(i, _):
        for s, buf, sem in ((0, buf0, sem0), (1, buf1, sem1)):
            c = 2 * i + s
            wait_out(buf, c - 2, sem)
            scatter_chunk(buf, c - 2, zeros)
            scatter_chunk(buf, c, ones)
            fire(buf, c, sem)
        return 0
    lax.fori_loop(1, NCHUNK // 2, pair_body, 0)

    wait_out(buf0, NCHUNK - 2, sem0)
    wait_out(buf1, NCHUNK - 1, sem1)


@jax.jit
def kernel(names):
    mesh = plsc.VectorSubcoreMesh(
        core_axis_name="c", subcore_axis_name="s",
        num_cores=NC, num_subcores=NS)
    out = pl.kernel(
        _body,
        out_type=jax.ShapeDtypeStruct((B * C,), jnp.float32),
        mesh=mesh,
        compiler_params=pltpu.CompilerParams(needs_layout_passes=False),
        scratch_types=[
            pltpu.VMEM((RPW * L,), jnp.int32),
            pltpu.VMEM((EPC,), jnp.int32),
            pltpu.VMEM((CB,), jnp.float32),
            pltpu.VMEM((CB,), jnp.float32),
            pltpu.SemaphoreType.DMA,
            pltpu.SemaphoreType.DMA,
            pltpu.SemaphoreType.DMA,
        ],
    )(names.reshape(-1))
    return out.reshape(B, C)


# speedup vs baseline: 30.3670x; 1.4378x over previous
import jax
import jax.numpy as jnp
from jax import lax
from jax.experimental import pallas as pl
from jax.experimental.pallas import tpu as pltpu
from jax.experimental.pallas import tpu_sc as plsc

B, L, C = 16384, 50, 1000
NC, NS = 2, 16
NW = NC * NS
RPW = B // NW
R = 32
NCHUNK = RPW // R
EPC = R * L
VECS = EPC // 16

_ZOFFS = tuple(range(0, C - 16, 16)) + (C - 16,)


def _body(names_hbm, out_hbm, names_v, rowid_v, buf0, buf1, semn, sem0, sem1):
    wid = lax.axis_index("s") * NC + lax.axis_index("c")
    ebase = wid * (RPW * L)
    rbase = wid * RPW

    names_cp = pltpu.async_copy(
        names_hbm.at[pl.ds(ebase, RPW * L)], names_v, semn)

    zeros = jnp.zeros((16,), jnp.float32)
    ones = jnp.full((16,), 1.0, jnp.float32)
    iota = lax.iota(jnp.int32, 16)

    def zero_row(r, _):
        for off in _ZOFFS:
            buf0[r, pl.ds(off, 16)] = zeros
            buf1[r, pl.ds(off, 16)] = zeros
        return 0
    lax.fori_loop(0, R, zero_row, 0)

    def rid_body(j, _):
        e = j * 16 + iota
        rowid_v[pl.ds(j * 16, 16)] = e // L
        return 0
    lax.fori_loop(0, VECS, rid_body, 0, unroll=4)

    names_cp.wait()

    def scatter_chunk(buf, c, val_vec):
        coff = c * EPC

        def s_body(j, _):
            nv = names_v[pl.ds(coff + j * 16, 16)]
            rv = rowid_v[pl.ds(j * 16, 16)]
            plsc.store_scatter(buf, [rv, nv], val_vec)
            return 0
        lax.fori_loop(0, VECS, s_body, 0, unroll=4)

    def fire(buf, c, sem):
        pltpu.async_copy(buf, out_hbm.at[pl.ds(rbase + c * R, R)], sem)

    def wait_out(buf, c, sem):
        pltpu.make_async_copy(
            buf, out_hbm.at[pl.ds(rbase + c * R, R)], sem).wait()

    scatter_chunk(buf0, 0, ones)
    fire(buf0, 0, sem0)
    scatter_chunk(buf1, 1, ones)
    fire(buf1, 1, sem1)

    def pair_body(i, _):
        for s, buf, sem in ((0, buf0, sem0), (1, buf1, sem1)):
            c = 2 * i + s
            wait_out(buf, c - 2, sem)
            scatter_chunk(buf, c - 2, zeros)
            scatter_chunk(buf, c, ones)
            fire(buf, c, sem)
        return 0
    lax.fori_loop(1, NCHUNK // 2, pair_body, 0)

    wait_out(buf0, NCHUNK - 2, sem0)
    wait_out(buf1, NCHUNK - 1, sem1)


@jax.jit
def kernel(names):
    mesh = plsc.VectorSubcoreMesh(
        core_axis_name="c", subcore_axis_name="s",
        num_cores=NC, num_subcores=NS)
    return pl.kernel(
        _body,
        out_type=jax.ShapeDtypeStruct((B, C), jnp.float32),
        mesh=mesh,
        compiler_params=pltpu.CompilerParams(needs_layout_passes=False),
        scratch_types=[
            pltpu.VMEM((RPW * L,), jnp.int32),
            pltpu.VMEM((EPC,), jnp.int32),
            pltpu.VMEM((R, C), jnp.float32),
            pltpu.VMEM((R, C), jnp.float32),
            pltpu.SemaphoreType.DMA,
            pltpu.SemaphoreType.DMA,
            pltpu.SemaphoreType.DMA,
        ],
    )(names.reshape(-1))


# speedup vs baseline: 32.5543x; 1.0720x over previous
import jax
import jax.numpy as jnp
from jax import lax
from jax.experimental import pallas as pl
from jax.experimental.pallas import tpu as pltpu
from jax.experimental.pallas import tpu_sc as plsc

B, L, C = 16384, 50, 1000
NC, NS = 2, 16
NW = NC * NS
NQ = 4
NT = NW // NQ
CC = 256
RB = 128
NBLK = B // (NT * RB)
NRING = 4


def _body(names_hbm, out_hbm, nm0, nm1, nm2, nm3, bufa, bufb,
          sn0, sn1, sn2, sn3, soa, sob):
    wid = lax.axis_index("s") * NC + lax.axis_index("c")
    team = wid // NQ
    q = wid % NQ
    c0 = jnp.where(q == NQ - 1, C - CC, q * CC)

    nslots = (nm0, nm1, nm2, nm3)
    nsems = (sn0, sn1, sn2, sn3)

    def r0_of(j):
        return (team * NBLK + j) * RB

    def names_cp(j):
        return pltpu.make_async_copy(
            names_hbm.at[:, pl.ds(r0_of(j), RB)],
            nslots[j % NRING], nsems[j % NRING])

    def out_cp(buf, j, sem):
        return pltpu.make_async_copy(
            buf, out_hbm.at[pl.ds(c0, CC), pl.ds(r0_of(j), RB)], sem)

    for j in range(NRING):
        names_cp(j).start()

    zeros = jnp.zeros((16,), jnp.float32)
    ones = jnp.full((16,), 1.0, jnp.float32)
    iota = lax.iota(jnp.int32, 16)
    cc_u = jnp.uint32(CC)

    def zero_body(ci, _):
        for k in range(RB // 16):
            bufa[ci, pl.ds(k * 16, 16)] = zeros
            bufb[ci, pl.ds(k * 16, 16)] = zeros
        return 0
    lax.fori_loop(0, CC, zero_body, 0)

    def scan_pass(buf, nm, val_vec):
        def l_body(l, _):
            for rs in range(RB // 16):
                nv = nm[l, pl.ds(rs * 16, 16)]
                cv = nv - c0
                mask = cv.astype(jnp.uint32) < cc_u
                plsc.store_scatter(buf, [cv, rs * 16 + iota], val_vec,
                                   mask=mask)
            return 0
        lax.fori_loop(0, L, l_body, 0)

    for j in range(NBLK):
        buf, sem = (bufa, soa) if j % 2 == 0 else (bufb, sob)
        if j >= 2:
            out_cp(buf, j - 2, sem).wait()
            scan_pass(buf, nslots[(j - 2) % NRING], zeros)
            if j + 2 < NBLK:
                names_cp(j + 2).start()
        names_cp(j).wait()
        scan_pass(buf, nslots[j % NRING], ones)
        out_cp(buf, j, sem).start()

    out_cp(bufa, NBLK - 2, soa).wait()
    out_cp(bufb, NBLK - 1, sob).wait()


@jax.jit
def kernel(names):
    mesh = plsc.VectorSubcoreMesh(
        core_axis_name="c", subcore_axis_name="s",
        num_cores=NC, num_subcores=NS)
    out_t = pl.kernel(
        _body,
        out_type=jax.ShapeDtypeStruct((C, B), jnp.float32),
        mesh=mesh,
        compiler_params=pltpu.CompilerParams(needs_layout_passes=False),
        scratch_types=[
            pltpu.VMEM((L, RB), jnp.int32),
            pltpu.VMEM((L, RB), jnp.int32),
            pltpu.VMEM((L, RB), jnp.int32),
            pltpu.VMEM((L, RB), jnp.int32),
            pltpu.VMEM((CC, RB), jnp.float32),
            pltpu.VMEM((CC, RB), jnp.float32),
            pltpu.SemaphoreType.DMA,
            pltpu.SemaphoreType.DMA,
            pltpu.SemaphoreType.DMA,
            pltpu.SemaphoreType.DMA,
            pltpu.SemaphoreType.DMA,
            pltpu.SemaphoreType.DMA,
        ],
    )(names.T)
    return out_t.T


# speedup vs baseline: 60.8092x; 1.8679x over previous
import jax
import jax.numpy as jnp
from jax import lax
from jax.experimental import pallas as pl
from jax.experimental.pallas import tpu as pltpu
from jax.experimental.pallas import tpu_sc as plsc

B, L, C = 16384, 50, 1000
NC, NS = 2, 16
NW = NC * NS
NQ = 4
NT = NW // NQ
CC = 256
RB = 128
NBLK = B // (NT * RB)
NRING = 4


def _body(names_hbm, out_hbm, nm0, nm1, nm2, nm3, bufa, bufb,
          sn0, sn1, sn2, sn3, soa, sob):
    wid = lax.axis_index("s") * NC + lax.axis_index("c")
    team = wid // NQ
    q = wid % NQ
    c0 = jnp.where(q == NQ - 1, C - CC, q * CC)

    nslots = (nm0, nm1, nm2, nm3)
    nsems = (sn0, sn1, sn2, sn3)

    def r0_of(j):
        return (team * NBLK + j) * RB

    def names_cp(j):
        return pltpu.make_async_copy(
            names_hbm.at[:, pl.ds(r0_of(j), RB)],
            nslots[j % NRING], nsems[j % NRING])

    def out_cp(buf, j, sem):
        return pltpu.make_async_copy(
            buf, out_hbm.at[pl.ds(c0, CC), pl.ds(r0_of(j), RB)], sem)

    for j in range(NRING):
        names_cp(j).start()

    zeros = jnp.zeros((16,), jnp.float32)
    ones = jnp.full((16,), 1.0, jnp.float32)
    iota = lax.iota(jnp.int32, 16)
    cc_u = jnp.uint32(CC)

    def zero_body(ci, _):
        for k in range(RB // 16):
            bufa[ci, pl.ds(k * 16, 16)] = zeros
            bufb[ci, pl.ds(k * 16, 16)] = zeros
        return 0
    lax.fori_loop(0, CC, zero_body, 0)

    def scan_pass(buf, nm, val_vec):
        @plsc.parallel_loop(0, L, unroll=2)
        def _(l):
            for rs in range(RB // 16):
                nv = nm[l, pl.ds(rs * 16, 16)]
                cv = nv - c0
                mask = cv.astype(jnp.uint32) < cc_u
                plsc.store_scatter(buf, [cv, rs * 16 + iota], val_vec,
                                   mask=mask)

    for j in range(NBLK):
        buf, sem = (bufa, soa) if j % 2 == 0 else (bufb, sob)
        if j >= 2:
            out_cp(buf, j - 2, sem).wait()
            scan_pass(buf, nslots[(j - 2) % NRING], zeros)
            if j + 2 < NBLK:
                names_cp(j + 2).start()
        names_cp(j).wait()
        scan_pass(buf, nslots[j % NRING], ones)
        out_cp(buf, j, sem).start()

    out_cp(bufa, NBLK - 2, soa).wait()
    out_cp(bufb, NBLK - 1, sob).wait()


@jax.jit
def kernel(names):
    mesh = plsc.VectorSubcoreMesh(
        core_axis_name="c", subcore_axis_name="s",
        num_cores=NC, num_subcores=NS)
    out_t = pl.kernel(
        _body,
        out_type=jax.ShapeDtypeStruct((C, B), jnp.float32),
        mesh=mesh,
        compiler_params=pltpu.CompilerParams(needs_layout_passes=False),
        scratch_types=[
            pltpu.VMEM((L, RB), jnp.int32),
            pltpu.VMEM((L, RB), jnp.int32),
            pltpu.VMEM((L, RB), jnp.int32),
            pltpu.VMEM((L, RB), jnp.int32),
            pltpu.VMEM((CC, RB), jnp.float32),
            pltpu.VMEM((CC, RB), jnp.float32),
            pltpu.SemaphoreType.DMA,
            pltpu.SemaphoreType.DMA,
            pltpu.SemaphoreType.DMA,
            pltpu.SemaphoreType.DMA,
            pltpu.SemaphoreType.DMA,
            pltpu.SemaphoreType.DMA,
        ],
    )(names.T)
    return out_t.T
